# baseline (speedup 1.0000x reference)
import jax
import jax.numpy as jnp
from jax.experimental import pallas as pl
from jax.experimental.pallas import tpu as pltpu

N_CLASSES = 100000
N_FEAT = 128
BLOCK_C = 16384
NBUF = 4

_N_FULL = N_CLASSES // BLOCK_C
_TAIL = N_CLASSES - _N_FULL * BLOCK_C
_GRID = _N_FULL + (1 if _TAIL else 0)


def _body(x_ref, b_ref, w_hbm, o_ref, wbuf, sems):
    i = pl.program_id(0)

    def start_full(chunk, slot):
        pltpu.make_async_copy(
            w_hbm.at[pl.ds(chunk * BLOCK_C, BLOCK_C)],
            wbuf.at[slot],
            sems.at[slot],
        ).start()

    def start_tail(slot):
        pltpu.make_async_copy(
            w_hbm.at[pl.ds(_N_FULL * BLOCK_C, _TAIL)],
            wbuf.at[slot, pl.ds(0, _TAIL)],
            sems.at[slot],
        ).start()

    @pl.when(i == 0)
    def _():
        for j in range(min(NBUF, _GRID)):
            if j < _N_FULL:
                start_full(j, j)
            elif _TAIL:
                start_tail(j)

    slot = jax.lax.rem(i, NBUF)

    if _TAIL:
        @pl.when(i < _N_FULL)
        def _():
            pltpu.make_async_copy(
                w_hbm.at[pl.ds(0, BLOCK_C)], wbuf.at[slot], sems.at[slot]
            ).wait()

        @pl.when(i == _N_FULL)
        def _():
            pltpu.make_async_copy(
                w_hbm.at[pl.ds(0, _TAIL)],
                wbuf.at[slot, pl.ds(0, _TAIL)],
                sems.at[slot],
            ).wait()
    else:
        pltpu.make_async_copy(
            w_hbm.at[pl.ds(0, BLOCK_C)], wbuf.at[slot], sems.at[slot]
        ).wait()

    acc = jax.lax.dot_general(
        x_ref[...], wbuf[slot],
        dimension_numbers=(((1,), (1,)), ((), ())),
        preferred_element_type=jnp.float32,
    )
    o_ref[...] = acc + b_ref[...]

    nxt = i + NBUF
    @pl.when(nxt < _N_FULL)
    def _():
        start_full(nxt, slot)

    if _TAIL:
        @pl.when(nxt == _N_FULL)
        def _():
            start_tail(slot)


def kernel(x, weight, bias):
    bias2d = bias.reshape(1, N_CLASSES)
    out = pl.pallas_call(
        _body,
        grid=(_GRID,),
        in_specs=[
            pl.BlockSpec((x.shape[0], N_FEAT), lambda i: (0, 0)),
            pl.BlockSpec((1, BLOCK_C), lambda i: (0, i)),
            pl.BlockSpec(memory_space=pltpu.MemorySpace.HBM),
        ],
        out_specs=pl.BlockSpec((x.shape[0], BLOCK_C), lambda i: (0, i)),
        out_shape=jax.ShapeDtypeStruct((x.shape[0], N_CLASSES), jnp.float32),
        scratch_shapes=[
            pltpu.VMEM((NBUF, BLOCK_C, N_FEAT), jnp.float32),
            pltpu.SemaphoreType.DMA((NBUF,)),
        ],
        compiler_params=pltpu.CompilerParams(
            dimension_semantics=("arbitrary",),
        ),
    )(x, bias2d, weight)
    return out


# speedup vs baseline: 1.1704x; 1.1704x over previous
import jax
import jax.numpy as jnp
from jax.experimental import pallas as pl
from jax.experimental.pallas import tpu as pltpu

N_CLASSES = 100000
N_FEAT = 128
BLOCK_C = 8192
NSTREAM = 2

_STEP_C = NSTREAM * BLOCK_C
_GRID = -(-N_CLASSES // _STEP_C)
_LAST_VALID = (N_CLASSES - 1) // BLOCK_C


def _body(x_ref, b_ref, *rest):
    w_refs = rest[:NSTREAM]
    o_ref = rest[NSTREAM]
    x = x_ref[...]
    for s in range(NSTREAM):
        acc = jax.lax.dot_general(
            x, w_refs[s][...],
            dimension_numbers=(((1,), (1,)), ((), ())),
            preferred_element_type=jnp.float32,
        )
        sl = pl.ds(s * BLOCK_C, BLOCK_C)
        o_ref[:, sl] = acc + b_ref[:, sl]


def _w_index_map(s):
    def index_map(i):
        return (jnp.minimum(i * NSTREAM + s, _LAST_VALID), 0)
    return index_map


def kernel(x, weight, bias):
    bias2d = bias.reshape(1, N_CLASSES)
    in_specs = [
        pl.BlockSpec((x.shape[0], N_FEAT), lambda i: (0, 0)),
        pl.BlockSpec((1, _STEP_C), lambda i: (0, i)),
    ] + [
        pl.BlockSpec((BLOCK_C, N_FEAT), _w_index_map(s)) for s in range(NSTREAM)
    ]
    out = pl.pallas_call(
        _body,
        grid=(_GRID,),
        in_specs=in_specs,
        out_specs=pl.BlockSpec((x.shape[0], _STEP_C), lambda i: (0, i)),
        out_shape=jax.ShapeDtypeStruct((x.shape[0], N_CLASSES), jnp.float32),
        compiler_params=pltpu.CompilerParams(
            dimension_semantics=("parallel",),
        ),
    )(x, bias2d, *([weight] * NSTREAM))
    return out
